# Initial kernel scaffold; baseline (speedup 1.0000x reference)
import jax
import jax.numpy as jnp
from jax.experimental import pallas as pl


def kernel(geo_feat, sem_feat, rsa_feat, pos, params):
    raise NotImplementedError("write your pallas kernel here")


# speedup vs baseline: 5.5437x; 5.5437x over previous
import functools
import math

import jax
import jax.numpy as jnp
from jax.experimental import pallas as pl

L = 2048
GEO = 1536
SEM = 512
RSA = 64
OUT = 256
H = 8
DH = 32
K = 10
BLK = 256
GRID = L // BLK


def _ln(x, g, b):
    m = jnp.mean(x, axis=-1, keepdims=True)
    v = jnp.mean((x - m) ** 2, axis=-1, keepdims=True)
    return (x - m) / jnp.sqrt(v + 1e-5) * g + b


def _leaky(x):
    return jnp.where(x >= 0, x, 0.01 * x)


def _projqkv_body(geo_ref, sem_ref, rsa_ref,
                  geo_w, geo_b, geo_g, geo_bb,
                  sem_w, sem_b, sem_g, sem_bb,
                  rsa_w, rsa_b, rsa_g, rsa_bb,
                  rt_w, rt_b, rt_g, rt_bb,
                  wqkv, bqkv,
                  geo_p_o, sem_p_o, rsa_o_o, geo_qkv_o, sem_qkv_o):
    f32 = jnp.float32
    geo_p = _ln(jnp.dot(geo_ref[...], geo_w[...], preferred_element_type=f32)
                + geo_b[...], geo_g[...], geo_bb[...])
    sem_p = _ln(jnp.dot(sem_ref[...], sem_w[...], preferred_element_type=f32)
                + sem_b[...], sem_g[...], sem_bb[...])
    rsa_p = _ln(jnp.dot(rsa_ref[...], rsa_w[...], preferred_element_type=f32)
                + rsa_b[...], rsa_g[...], rsa_bb[...])
    rsa_o_o[...] = _leaky(_ln(
        jnp.dot(rsa_p, rt_w[...], preferred_element_type=f32) + rt_b[...],
        rt_g[...], rt_bb[...]))
    geo_p_o[...] = geo_p
    sem_p_o[...] = sem_p
    geo_qkv_o[...] = jnp.dot(geo_p, wqkv[...], preferred_element_type=f32) + bqkv[...]
    sem_qkv_o[...] = jnp.dot(sem_p, wqkv[...], preferred_element_type=f32) + bqkv[...]


def _attend(q, kf, vf, mask):
    scale = 1.0 / math.sqrt(DH)
    outs = []
    for h in range(H):
        qh = q[:, h * DH:(h + 1) * DH]
        kh = kf[:, h * DH:(h + 1) * DH]
        vh = vf[:, h * DH:(h + 1) * DH]
        s = jax.lax.dot_general(qh, kh, (((1,), (1,)), ((), ())),
                                preferred_element_type=jnp.float32) * scale
        s = jnp.where(mask, s, -jnp.inf)
        m = jnp.max(s, axis=1, keepdims=True)
        p = jnp.exp(s - m)
        denom = jnp.sum(p, axis=1, keepdims=True)
        a = p / denom
        outs.append(jnp.dot(a, vh, preferred_element_type=jnp.float32))
    return jnp.concatenate(outs, axis=1)


def _attn_body(pos_ref, pos_t_ref, geo_qkv_ref, sem_qkv_ref,
               attn_geo_o, attn_sem_o):
    i = pl.program_id(0)
    r0 = i * BLK
    rows = pl.ds(r0, BLK)

    x_r = pos_ref[rows, 0:1]
    y_r = pos_ref[rows, 1:2]
    z_r = pos_ref[rows, 2:3]
    x_c = pos_t_ref[0:1, :]
    y_c = pos_t_ref[1:2, :]
    z_c = pos_t_ref[2:3, :]
    sq_r = x_r * x_r + y_r * y_r + z_r * z_r
    sq_c = x_c * x_c + y_c * y_c + z_c * z_c
    dot_rc = x_r * x_c + y_r * y_c + z_r * z_c
    d2 = sq_r + sq_c - 2.0 * dot_rc
    d = jnp.sqrt(jnp.maximum(d2, 0.0))

    colid = jax.lax.broadcasted_iota(jnp.int32, (BLK, L), 1)
    mask = jnp.zeros((BLK, L), dtype=jnp.bool_)
    for _ in range(K):
        m = jnp.min(d, axis=1, keepdims=True)
        cand = jnp.where(d == m, colid, jnp.int32(L))
        imin = jnp.min(cand, axis=1, keepdims=True)
        sel = colid == imin
        mask = jnp.logical_or(mask, sel)
        d = jnp.where(sel, jnp.inf, d)

    q_geo = geo_qkv_ref[rows, 0:OUT]
    k_sem = sem_qkv_ref[:, OUT:2 * OUT]
    v_sem = sem_qkv_ref[:, 2 * OUT:3 * OUT]
    attn_geo_o[...] = _attend(q_geo, k_sem, v_sem, mask)

    q_sem = sem_qkv_ref[rows, 0:OUT]
    k_geo = geo_qkv_ref[:, OUT:2 * OUT]
    v_geo = geo_qkv_ref[:, 2 * OUT:3 * OUT]
    attn_sem_o[...] = _attend(q_sem, k_geo, v_geo, mask)


def _head_body(geo_p_ref, sem_p_ref, rsa_o_ref, attn_geo_ref, attn_sem_ref,
               o_w, o_b, ab_ref, ln1_g, ln1_b, ln2_g, ln2_b,
               f1_w, f1_b, f1_g, f1_bb, f2_w, f2_b, f2_g, f2_bb,
               out_o):
    f32 = jnp.float32
    a_geo = ab_ref[0, 0]
    b_geo = ab_ref[0, 1]
    a_sem = ab_ref[0, 2]
    b_sem = ab_ref[0, 3]
    ag = jnp.dot(attn_geo_ref[...], o_w[...], preferred_element_type=f32) + o_b[...]
    geo_out = _ln(a_geo * geo_p_ref[...] + b_geo * ag, ln1_g[...], ln1_b[...])
    asm = jnp.dot(attn_sem_ref[...], o_w[...], preferred_element_type=f32) + o_b[...]
    sem_out = _ln(a_sem * sem_p_ref[...] + b_sem * asm, ln2_g[...], ln2_b[...])
    h = (jnp.dot(geo_out, f1_w[0:OUT, :], preferred_element_type=f32)
         + jnp.dot(sem_out, f1_w[OUT:2 * OUT, :], preferred_element_type=f32)
         + jnp.dot(rsa_o_ref[...], f1_w[2 * OUT:3 * OUT, :], preferred_element_type=f32)
         + f1_b[...])
    h = _leaky(_ln(h, f1_g[...], f1_bb[...]))
    f = _leaky(_ln(jnp.dot(h, f2_w[...], preferred_element_type=f32) + f2_b[...],
                   f2_g[...], f2_bb[...]))
    out_o[...] = f


def _row_spec(d):
    return pl.BlockSpec((BLK, d), lambda i: (i, 0))


def _full_spec(shape):
    n = len(shape)
    return pl.BlockSpec(shape, lambda i: (0,) * n)


@jax.jit
def kernel(geo_feat, sem_feat, rsa_feat, pos, params):
    p = params
    f32 = jnp.float32
    row = lambda v: jnp.reshape(v, (1, -1)).astype(f32)

    wqkv = jnp.concatenate([p['q_w'], p['k_w'], p['v_w']], axis=1)
    bqkv = jnp.concatenate([p['q_b'], p['k_b'], p['v_b']]).reshape(1, -1)

    geo_p, sem_p, rsa_out, geo_qkv, sem_qkv = pl.pallas_call(
        _projqkv_body,
        grid=(GRID,),
        in_specs=[
            _row_spec(GEO), _row_spec(SEM), _row_spec(RSA),
            _full_spec((GEO, OUT)), _full_spec((1, OUT)), _full_spec((1, OUT)), _full_spec((1, OUT)),
            _full_spec((SEM, OUT)), _full_spec((1, OUT)), _full_spec((1, OUT)), _full_spec((1, OUT)),
            _full_spec((RSA, OUT)), _full_spec((1, OUT)), _full_spec((1, OUT)), _full_spec((1, OUT)),
            _full_spec((OUT, OUT)), _full_spec((1, OUT)), _full_spec((1, OUT)), _full_spec((1, OUT)),
            _full_spec((OUT, 3 * OUT)), _full_spec((1, 3 * OUT)),
        ],
        out_specs=[_row_spec(OUT), _row_spec(OUT), _row_spec(OUT),
                   _row_spec(3 * OUT), _row_spec(3 * OUT)],
        out_shape=[
            jax.ShapeDtypeStruct((L, OUT), f32),
            jax.ShapeDtypeStruct((L, OUT), f32),
            jax.ShapeDtypeStruct((L, OUT), f32),
            jax.ShapeDtypeStruct((L, 3 * OUT), f32),
            jax.ShapeDtypeStruct((L, 3 * OUT), f32),
        ],
    )(geo_feat, sem_feat, rsa_feat,
      p['geo_w'], row(p['geo_b']), row(p['geo_g']), row(p['geo_bb']),
      p['sem_w'], row(p['sem_b']), row(p['sem_g']), row(p['sem_bb']),
      p['rsa_w'], row(p['rsa_b']), row(p['rsa_g']), row(p['rsa_bb']),
      p['rt_w'], row(p['rt_b']), row(p['rt_g']), row(p['rt_bb']),
      wqkv, bqkv)

    pos_t = pos.T
    attn_geo, attn_sem = pl.pallas_call(
        _attn_body,
        grid=(GRID,),
        in_specs=[_full_spec((L, 3)), _full_spec((3, L)),
                  _full_spec((L, 3 * OUT)), _full_spec((L, 3 * OUT))],
        out_specs=[_row_spec(OUT), _row_spec(OUT)],
        out_shape=[jax.ShapeDtypeStruct((L, OUT), f32),
                   jax.ShapeDtypeStruct((L, OUT), f32)],
    )(pos, pos_t, geo_qkv, sem_qkv)

    ab = jnp.stack([p['a_geo'], p['b_geo'], p['a_sem'], p['b_sem']]).reshape(1, 4)
    fused = pl.pallas_call(
        _head_body,
        grid=(GRID,),
        in_specs=[_row_spec(OUT), _row_spec(OUT), _row_spec(OUT),
                  _row_spec(OUT), _row_spec(OUT),
                  _full_spec((OUT, OUT)), _full_spec((1, OUT)),
                  _full_spec((1, 4)),
                  _full_spec((1, OUT)), _full_spec((1, OUT)),
                  _full_spec((1, OUT)), _full_spec((1, OUT)),
                  _full_spec((3 * OUT, 2 * OUT)), _full_spec((1, 2 * OUT)),
                  _full_spec((1, 2 * OUT)), _full_spec((1, 2 * OUT)),
                  _full_spec((2 * OUT, OUT)), _full_spec((1, OUT)),
                  _full_spec((1, OUT)), _full_spec((1, OUT))],
        out_specs=_row_spec(OUT),
        out_shape=jax.ShapeDtypeStruct((L, OUT), f32),
    )(geo_p, sem_p, rsa_out, attn_geo, attn_sem,
      p['o_w'], row(p['o_b']), ab,
      row(p['ln1_g']), row(p['ln1_b']), row(p['ln2_g']), row(p['ln2_b']),
      p['f1_w'], row(p['f1_b']), row(p['f1_g']), row(p['f1_bb']),
      p['f2_w'], row(p['f2_b']), row(p['f2_g']), row(p['f2_bb']))
    return fused
